# baseline (speedup 1.0000x reference)
import functools

import jax
import jax.numpy as jnp
from jax import lax
from jax.experimental import pallas as pl
from jax.experimental.pallas import tpu as pltpu
from jax.experimental.pallas import tpu_sc as plsc

N = 10000
E = 320000
DV = 128
DE = 16
DH = 64
D3 = 3 * DH

EDGE_BLOCK = 2000
NSLICE = 8
ES = E // NSLICE

SC_CORES = 2
SC_SUBCORES = 16
NW = SC_CORES * SC_SUBCORES
CH = 128
ZCH = 200
N_ZCH = N // ZCH


def _rn16(u):
    return u + jnp.uint32(0x8000)


def _proj_body(x_ref, w_ref, b_ref, o_ref):
    t = jnp.dot(x_ref[...], w_ref[...],
                preferred_element_type=jnp.float32) + b_ref[...]
    a = t[:, :DV]
    b = t[:, DV:]
    ba = lax.bitcast_convert_type(a, jnp.uint32)
    bb = lax.bitcast_convert_type(b, jnp.uint32)
    word = (_rn16(ba) >> 16) | (_rn16(bb) & jnp.uint32(0xFFFF0000))
    o_ref[...] = lax.bitcast_convert_type(word, jnp.float32)


def _node_tables(node_fea, Wqkv, bqkv):
    return pl.pallas_call(
        _proj_body,
        out_shape=jax.ShapeDtypeStruct((N, DV), jnp.float32),
    )(node_fea, Wqkv, bqkv)


def _make_gather_body(n_chunks):
    def _gather_body(tp_hbm, idx1_hbm, idx2_hbm, g1_hbm, g2_hbm,
                     idx1_v, idx2_v, r1_v, r2_v):
        c = lax.axis_index("c")
        s = lax.axis_index("s")
        wid = c * SC_SUBCORES + s
        n_iter = (n_chunks + NW - 1) // NW

        def _chunk(t, carry):
            cid = wid + t * NW

            @pl.when(cid < n_chunks)
            def _do():
                base = pl.multiple_of(cid * CH, 8)
                pltpu.sync_copy(idx1_hbm.at[pl.ds(base, CH)], idx1_v)
                pltpu.sync_copy(idx2_hbm.at[pl.ds(base, CH)], idx2_v)
                pltpu.sync_copy(tp_hbm.at[idx1_v], r1_v)
                pltpu.sync_copy(tp_hbm.at[idx2_v], r2_v)
                pltpu.sync_copy(r1_v, g1_hbm.at[pl.ds(base, CH)])
                pltpu.sync_copy(r2_v, g2_hbm.at[pl.ds(base, CH)])

            return carry

        lax.fori_loop(0, n_iter, _chunk, 0)

    return _gather_body


def _sc_gather(TP, idx1, idx2):
    ne = idx1.shape[0]
    mesh = plsc.VectorSubcoreMesh(core_axis_name="c", subcore_axis_name="s",
                                  num_cores=SC_CORES, num_subcores=SC_SUBCORES)
    return pl.kernel(
        _make_gather_body(ne // CH),
        out_type=(jax.ShapeDtypeStruct((ne, DV), jnp.float32),
                  jax.ShapeDtypeStruct((ne, DV), jnp.float32)),
        mesh=mesh,
        scratch_types=[
            pltpu.VMEM((CH,), jnp.int32),
            pltpu.VMEM((CH,), jnp.int32),
            pltpu.VMEM((CH, DV), jnp.float32),
            pltpu.VMEM((CH, DV), jnp.float32),
        ],
    )(TP, idx1, idx2)


def _ln(x, g, b):
    mu = jnp.mean(x, axis=-1, keepdims=True)
    xc = x - mu
    var = jnp.mean(xc * xc, axis=-1, keepdims=True)
    return xc * jax.lax.rsqrt(var + 1e-5) * g + b


def _edge_body(g1_ref, g2_ref, ef_ref, we_ref, be_ref, g1v_ref, b1v_ref,
               wu_ref, bu_ref, wm_ref, bm_ref, g2v_ref, b2v_ref, o_ref):
    w1 = lax.bitcast_convert_type(g1_ref[...], jnp.uint32)
    w2 = lax.bitcast_convert_type(g2_ref[...], jnp.uint32)
    hi_mask = jnp.uint32(0xFFFF0000)
    QK1 = lax.bitcast_convert_type(w1 << 16, jnp.float32)
    Vx1 = lax.bitcast_convert_type(w1 & hi_mask, jnp.float32)
    QK2 = lax.bitcast_convert_type(w2 << 16, jnp.float32)
    Vx2 = lax.bitcast_convert_type(w2 & hi_mask, jnp.float32)
    Q1 = QK1[:, 0:DH]
    K1 = QK1[:, DH:2 * DH]
    V1 = Vx1[:, 0:DH]
    K2 = QK2[:, DH:2 * DH]
    V2 = Vx2[:, 0:DH]
    EP = jnp.dot(ef_ref[...], we_ref[...],
                 preferred_element_type=jnp.float32) + be_ref[...]
    inv_sqrt = 1.0 / (D3 ** 0.5)
    aij = jnp.concatenate([Q1 * K1, Q1 * K2, Q1 * EP], axis=1) * inv_sqrt
    ln1 = _ln(aij, g1v_ref[...], b1v_ref[...])
    m1 = 1.0 / (1.0 + jnp.exp(-ln1))
    zij = jnp.concatenate([V1, V2, EP], axis=1).astype(jnp.bfloat16)
    m2 = jnp.dot(zij, wu_ref[...].astype(jnp.bfloat16),
                 preferred_element_type=jnp.float32) + bu_ref[...]
    mij = (m1 * m2).astype(jnp.bfloat16)
    t = jnp.dot(mij, wm_ref[...].astype(jnp.bfloat16),
                preferred_element_type=jnp.float32) + bm_ref[...]
    o_ref[...] = _ln(t, g2v_ref[...], b2v_ref[...])


def _edge_mlp(G1, G2, edge_fea, We, be, g1, b1, Wu, bu, Wm, bm, g2, b2):
    ne = G1.shape[0]
    nblk = ne // EDGE_BLOCK
    row_spec = lambda d: pl.BlockSpec((EDGE_BLOCK, d), lambda i: (i, 0))
    full = lambda a, b: pl.BlockSpec((a, b), lambda i: (0, 0))
    return pl.pallas_call(
        _edge_body,
        grid=(nblk,),
        in_specs=[
            row_spec(DV), row_spec(DV), row_spec(DE),
            full(DE, DH), full(1, DH), full(1, D3), full(1, D3),
            full(D3, D3), full(1, D3), full(D3, DV), full(1, DV),
            full(1, DV), full(1, DV),
        ],
        out_specs=row_spec(DV),
        out_shape=jax.ShapeDtypeStruct((ne, DV), jnp.float32),
    )(G1, G2, edge_fea, We, be.reshape(1, DH), g1.reshape(1, D3),
      b1.reshape(1, D3), Wu, bu.reshape(1, D3), Wm, bm.reshape(1, DV),
      g2.reshape(1, DV), b2.reshape(1, DV))


def _make_scatter_body(n_chunks):
    def _scatter_body(msg_hbm, idx_hbm, p0_hbm, p1_hbm,
                      idx_v, msg_v, stage_v, acc):
        c = lax.axis_index("c")
        s = lax.axis_index("s")
        wid = c * SC_SUBCORES + s

        z16 = jnp.zeros((16,), jnp.float32)

        def _zrow(i, carry):
            for j in range(DV // 16):
                stage_v[i, pl.ds(j * 16, 16)] = z16
            return carry

        lax.fori_loop(0, ZCH, _zrow, 0)

        def _zchunk(t, carry):
            zc = s + t * SC_SUBCORES

            @pl.when(zc < N_ZCH)
            def _do():
                rows = pl.ds(pl.multiple_of(zc * ZCH, 8), ZCH)
                pltpu.sync_copy(stage_v, acc.at[rows])

            return carry

        lax.fori_loop(0, (N_ZCH + SC_SUBCORES - 1) // SC_SUBCORES, _zchunk, 0)
        plsc.subcore_barrier()

        def _chunk(t, carry):
            cid = wid + t * NW

            @pl.when(cid < n_chunks)
            def _do():
                base = pl.multiple_of(cid * CH, 8)
                pltpu.sync_copy(idx_hbm.at[pl.ds(base, CH)], idx_v)
                pltpu.sync_copy(msg_hbm.at[pl.ds(base, CH)], msg_v)
                pltpu.sync_copy(msg_v, acc.at[idx_v], add=True)

            return carry

        lax.fori_loop(0, (n_chunks + NW - 1) // NW, _chunk, 0)
        plsc.subcore_barrier()

        def _dchunk(t, carry):
            zc = s + t * SC_SUBCORES

            @pl.when(zc < N_ZCH)
            def _do():
                rows = pl.ds(pl.multiple_of(zc * ZCH, 8), ZCH)
                pltpu.sync_copy(acc.at[rows], stage_v)

                @pl.when(c == 0)
                def _c0():
                    pltpu.sync_copy(stage_v, p0_hbm.at[rows])

                @pl.when(c == 1)
                def _c1():
                    pltpu.sync_copy(stage_v, p1_hbm.at[rows])

            return carry

        lax.fori_loop(0, (N_ZCH + SC_SUBCORES - 1) // SC_SUBCORES, _dchunk, 0)

    return _scatter_body


def _sc_scatter(msg, idx1):
    ne = idx1.shape[0]
    mesh = plsc.VectorSubcoreMesh(core_axis_name="c", subcore_axis_name="s",
                                  num_cores=SC_CORES, num_subcores=SC_SUBCORES)
    return pl.kernel(
        _make_scatter_body(ne // CH),
        out_type=(jax.ShapeDtypeStruct((N, DV), jnp.float32),
                  jax.ShapeDtypeStruct((N, DV), jnp.float32)),
        mesh=mesh,
        scratch_types=[
            pltpu.VMEM((CH,), jnp.int32),
            pltpu.VMEM((CH, DV), jnp.float32),
            pltpu.VMEM((ZCH, DV), jnp.float32),
            pltpu.VMEM_SHARED((N, DV), jnp.float32),
        ],
    )(msg, idx1)


def _add_body(*refs):
    o_ref = refs[-1]
    acc = refs[0][...]
    for r in refs[1:-1]:
        acc = acc + r[...]
    o_ref[...] = acc


def _add_partials(parts):
    spec = pl.BlockSpec((EDGE_BLOCK, DV), lambda i: (i, 0))
    return pl.pallas_call(
        _add_body,
        grid=(N // EDGE_BLOCK,),
        in_specs=[spec] * len(parts),
        out_specs=spec,
        out_shape=jax.ShapeDtypeStruct((N, DV), jnp.float32),
    )(*parts)


def kernel(node_fea, idx1, idx2, edge_fea, Wq, bq, Wk, bk, Wv, bv, We, be,
           g1, b1, Wu, bu, Wm, bm, g2, b2):
    Wqkv = jnp.concatenate(
        [Wq, Wk, Wv, jnp.zeros((DV, 2 * DV - D3), jnp.float32)], axis=1)
    bqkv = jnp.concatenate(
        [bq, bk, bv, jnp.zeros((2 * DV - D3,), jnp.float32)]).reshape(1, 2 * DV)
    TP = _node_tables(node_fea, Wqkv, bqkv)
    parts = []
    for si in range(NSLICE):
        sl = slice(si * ES, (si + 1) * ES)
        i1 = idx1[sl]
        G1, G2 = _sc_gather(TP, i1, idx2[sl])
        msg = _edge_mlp(G1, G2, edge_fea[sl], We, be, g1, b1,
                        Wu, bu, Wm, bm, g2, b2)
        p0, p1 = _sc_scatter(msg, i1)
        parts.extend([p0, p1])
    return _add_partials(parts)


# speedup vs baseline: 1.1904x; 1.1904x over previous
import functools

import jax
import jax.numpy as jnp
from jax import lax
from jax.experimental import pallas as pl
from jax.experimental.pallas import tpu as pltpu
from jax.experimental.pallas import tpu_sc as plsc

N = 10000
E = 320000
DV = 128
DE = 16
DH = 64
D3 = 3 * DH

EDGE_BLOCK = 2000
NSLICE = 4
ES = E // NSLICE

SC_CORES = 2
SC_SUBCORES = 16
NW = SC_CORES * SC_SUBCORES
CH = 128
ZCH = 200
N_ZCH = N // ZCH


def _rn16(u):
    return u + jnp.uint32(0x8000)


def _proj_body(x_ref, w_ref, b_ref, o_ref):
    t = jnp.dot(x_ref[...], w_ref[...],
                preferred_element_type=jnp.float32) + b_ref[...]
    a = t[:, :DV]
    b = t[:, DV:]
    ba = lax.bitcast_convert_type(a, jnp.uint32)
    bb = lax.bitcast_convert_type(b, jnp.uint32)
    word = (_rn16(ba) >> 16) | (_rn16(bb) & jnp.uint32(0xFFFF0000))
    o_ref[...] = lax.bitcast_convert_type(word, jnp.float32)


def _node_tables(node_fea, Wqkv, bqkv):
    return pl.pallas_call(
        _proj_body,
        out_shape=jax.ShapeDtypeStruct((N, DV), jnp.float32),
    )(node_fea, Wqkv, bqkv)


def _make_gather_body(n_chunks):
    def _gather_body(tp_hbm, idx1_hbm, idx2_hbm, g1_hbm, g2_hbm,
                     i1a, i2a, r1a, r2a, i1b, i2b, r1b, r2b, sema, semb):
        c = lax.axis_index("c")
        s = lax.axis_index("s")
        wid = c * SC_SUBCORES + s
        n_iter = (n_chunks + NW - 1) // NW
        n_pair = (n_iter + 1) // 2

        def _one(u, parity, i1v, i2v, r1v, r2v, sem):
            cid = wid + (2 * u + parity) * NW

            @pl.when(cid < n_chunks)
            def _do():
                base = pl.multiple_of(cid * CH, 8)

                @pl.when(u > 0)
                def _drain():
                    pltpu.make_async_copy(
                        r1v, g1_hbm.at[pl.ds(0, CH)], sem).wait()
                    pltpu.make_async_copy(
                        r2v, g2_hbm.at[pl.ds(0, CH)], sem).wait()

                pltpu.sync_copy(idx1_hbm.at[pl.ds(base, CH)], i1v)
                pltpu.sync_copy(idx2_hbm.at[pl.ds(base, CH)], i2v)
                pltpu.sync_copy(tp_hbm.at[i1v], r1v)
                pltpu.sync_copy(tp_hbm.at[i2v], r2v)
                pltpu.async_copy(r1v, g1_hbm.at[pl.ds(base, CH)], sem)
                pltpu.async_copy(r2v, g2_hbm.at[pl.ds(base, CH)], sem)

        def _pair(u, carry):
            _one(u, 0, i1a, i2a, r1a, r2a, sema)
            _one(u, 1, i1b, i2b, r1b, r2b, semb)
            return carry

        lax.fori_loop(0, n_pair, _pair, 0)
        pltpu.make_async_copy(r1a, g1_hbm.at[pl.ds(0, CH)], sema).wait()
        pltpu.make_async_copy(r2a, g2_hbm.at[pl.ds(0, CH)], sema).wait()
        pltpu.make_async_copy(r1b, g1_hbm.at[pl.ds(0, CH)], semb).wait()
        pltpu.make_async_copy(r2b, g2_hbm.at[pl.ds(0, CH)], semb).wait()

    return _gather_body


def _sc_gather(TP, idx1, idx2):
    ne = idx1.shape[0]
    mesh = plsc.VectorSubcoreMesh(core_axis_name="c", subcore_axis_name="s",
                                  num_cores=SC_CORES, num_subcores=SC_SUBCORES)
    return pl.kernel(
        _make_gather_body(ne // CH),
        out_type=(jax.ShapeDtypeStruct((ne, DV), jnp.float32),
                  jax.ShapeDtypeStruct((ne, DV), jnp.float32)),
        mesh=mesh,
        scratch_types=[
            pltpu.VMEM((CH,), jnp.int32),
            pltpu.VMEM((CH,), jnp.int32),
            pltpu.VMEM((CH, DV), jnp.float32),
            pltpu.VMEM((CH, DV), jnp.float32),
            pltpu.VMEM((CH,), jnp.int32),
            pltpu.VMEM((CH,), jnp.int32),
            pltpu.VMEM((CH, DV), jnp.float32),
            pltpu.VMEM((CH, DV), jnp.float32),
            pltpu.SemaphoreType.DMA,
            pltpu.SemaphoreType.DMA,
        ],
    )(TP, idx1, idx2)


def _ln(x, g, b):
    mu = jnp.mean(x, axis=-1, keepdims=True)
    xc = x - mu
    var = jnp.mean(xc * xc, axis=-1, keepdims=True)
    return xc * jax.lax.rsqrt(var + 1e-5) * g + b


def _edge_body(g1_ref, g2_ref, ef_ref, we_ref, be_ref, g1v_ref, b1v_ref,
               wu_ref, bu_ref, wm_ref, bm_ref, g2v_ref, b2v_ref, o_ref):
    w1 = lax.bitcast_convert_type(g1_ref[...], jnp.uint32)
    w2 = lax.bitcast_convert_type(g2_ref[...], jnp.uint32)
    hi_mask = jnp.uint32(0xFFFF0000)
    QK1 = lax.bitcast_convert_type(w1 << 16, jnp.float32)
    Vx1 = lax.bitcast_convert_type(w1 & hi_mask, jnp.float32)
    QK2 = lax.bitcast_convert_type(w2 << 16, jnp.float32)
    Vx2 = lax.bitcast_convert_type(w2 & hi_mask, jnp.float32)
    Q1 = QK1[:, 0:DH]
    K1 = QK1[:, DH:2 * DH]
    V1 = Vx1[:, 0:DH]
    K2 = QK2[:, DH:2 * DH]
    V2 = Vx2[:, 0:DH]
    EP = jnp.dot(ef_ref[...], we_ref[...],
                 preferred_element_type=jnp.float32) + be_ref[...]
    inv_sqrt = 1.0 / (D3 ** 0.5)
    aij = jnp.concatenate([Q1 * K1, Q1 * K2, Q1 * EP], axis=1) * inv_sqrt
    ln1 = _ln(aij, g1v_ref[...], b1v_ref[...])
    m1 = 1.0 / (1.0 + jnp.exp(-ln1))
    zij = jnp.concatenate([V1, V2, EP], axis=1).astype(jnp.bfloat16)
    m2 = jnp.dot(zij, wu_ref[...].astype(jnp.bfloat16),
                 preferred_element_type=jnp.float32) + bu_ref[...]
    mij = (m1 * m2).astype(jnp.bfloat16)
    t = jnp.dot(mij, wm_ref[...].astype(jnp.bfloat16),
                preferred_element_type=jnp.float32) + bm_ref[...]
    o_ref[...] = _ln(t, g2v_ref[...], b2v_ref[...])


def _edge_mlp(G1, G2, edge_fea, We, be, g1, b1, Wu, bu, Wm, bm, g2, b2):
    ne = G1.shape[0]
    nblk = ne // EDGE_BLOCK
    row_spec = lambda d: pl.BlockSpec((EDGE_BLOCK, d), lambda i: (i, 0))
    full = lambda a, b: pl.BlockSpec((a, b), lambda i: (0, 0))
    return pl.pallas_call(
        _edge_body,
        grid=(nblk,),
        in_specs=[
            row_spec(DV), row_spec(DV), row_spec(DE),
            full(DE, DH), full(1, DH), full(1, D3), full(1, D3),
            full(D3, D3), full(1, D3), full(D3, DV), full(1, DV),
            full(1, DV), full(1, DV),
        ],
        out_specs=row_spec(DV),
        out_shape=jax.ShapeDtypeStruct((ne, DV), jnp.float32),
    )(G1, G2, edge_fea, We, be.reshape(1, DH), g1.reshape(1, D3),
      b1.reshape(1, D3), Wu, bu.reshape(1, D3), Wm, bm.reshape(1, DV),
      g2.reshape(1, DV), b2.reshape(1, DV))


def _make_scatter_body(n_chunks):
    def _scatter_body(msg_hbm, idx_hbm, p0_hbm, p1_hbm,
                      idx_v, msg_v, stage_v, acc):
        c = lax.axis_index("c")
        s = lax.axis_index("s")
        wid = c * SC_SUBCORES + s

        z16 = jnp.zeros((16,), jnp.float32)

        def _zrow(i, carry):
            for j in range(DV // 16):
                stage_v[i, pl.ds(j * 16, 16)] = z16
            return carry

        lax.fori_loop(0, ZCH, _zrow, 0)

        def _zchunk(t, carry):
            zc = s + t * SC_SUBCORES

            @pl.when(zc < N_ZCH)
            def _do():
                rows = pl.ds(pl.multiple_of(zc * ZCH, 8), ZCH)
                pltpu.sync_copy(stage_v, acc.at[rows])

            return carry

        lax.fori_loop(0, (N_ZCH + SC_SUBCORES - 1) // SC_SUBCORES, _zchunk, 0)
        plsc.subcore_barrier()

        def _chunk(t, carry):
            cid = wid + t * NW

            @pl.when(cid < n_chunks)
            def _do():
                base = pl.multiple_of(cid * CH, 8)
                pltpu.sync_copy(idx_hbm.at[pl.ds(base, CH)], idx_v)
                pltpu.sync_copy(msg_hbm.at[pl.ds(base, CH)], msg_v)
                pltpu.sync_copy(msg_v, acc.at[idx_v], add=True)

            return carry

        lax.fori_loop(0, (n_chunks + NW - 1) // NW, _chunk, 0)
        plsc.subcore_barrier()

        def _dchunk(t, carry):
            zc = s + t * SC_SUBCORES

            @pl.when(zc < N_ZCH)
            def _do():
                rows = pl.ds(pl.multiple_of(zc * ZCH, 8), ZCH)
                pltpu.sync_copy(acc.at[rows], stage_v)

                @pl.when(c == 0)
                def _c0():
                    pltpu.sync_copy(stage_v, p0_hbm.at[rows])

                @pl.when(c == 1)
                def _c1():
                    pltpu.sync_copy(stage_v, p1_hbm.at[rows])

            return carry

        lax.fori_loop(0, (N_ZCH + SC_SUBCORES - 1) // SC_SUBCORES, _dchunk, 0)

    return _scatter_body


def _sc_scatter(msg, idx1):
    ne = idx1.shape[0]
    mesh = plsc.VectorSubcoreMesh(core_axis_name="c", subcore_axis_name="s",
                                  num_cores=SC_CORES, num_subcores=SC_SUBCORES)
    return pl.kernel(
        _make_scatter_body(ne // CH),
        out_type=(jax.ShapeDtypeStruct((N, DV), jnp.float32),
                  jax.ShapeDtypeStruct((N, DV), jnp.float32)),
        mesh=mesh,
        scratch_types=[
            pltpu.VMEM((CH,), jnp.int32),
            pltpu.VMEM((CH, DV), jnp.float32),
            pltpu.VMEM((ZCH, DV), jnp.float32),
            pltpu.VMEM_SHARED((N, DV), jnp.float32),
        ],
    )(msg, idx1)


def _add_body(*refs):
    o_ref = refs[-1]
    acc = refs[0][...]
    for r in refs[1:-1]:
        acc = acc + r[...]
    o_ref[...] = acc


def _add_partials(parts):
    spec = pl.BlockSpec((EDGE_BLOCK, DV), lambda i: (i, 0))
    return pl.pallas_call(
        _add_body,
        grid=(N // EDGE_BLOCK,),
        in_specs=[spec] * len(parts),
        out_specs=spec,
        out_shape=jax.ShapeDtypeStruct((N, DV), jnp.float32),
    )(*parts)


def kernel(node_fea, idx1, idx2, edge_fea, Wq, bq, Wk, bk, Wv, bv, We, be,
           g1, b1, Wu, bu, Wm, bm, g2, b2):
    Wqkv = jnp.concatenate(
        [Wq, Wk, Wv, jnp.zeros((DV, 2 * DV - D3), jnp.float32)], axis=1)
    bqkv = jnp.concatenate(
        [bq, bk, bv, jnp.zeros((2 * DV - D3,), jnp.float32)]).reshape(1, 2 * DV)
    TP = _node_tables(node_fea, Wqkv, bqkv)
    parts = []
    for si in range(NSLICE):
        sl = slice(si * ES, (si + 1) * ES)
        i1 = idx1[sl]
        G1, G2 = _sc_gather(TP, i1, idx2[sl])
        msg = _edge_mlp(G1, G2, edge_fea[sl], We, be, g1, b1,
                        Wu, bu, Wm, bm, g2, b2)
        p0, p1 = _sc_scatter(msg, i1)
        parts.extend([p0, p1])
    return _add_partials(parts)


# speedup vs baseline: 1.2325x; 1.0354x over previous
import functools

import jax
import jax.numpy as jnp
from jax import lax
from jax.experimental import pallas as pl
from jax.experimental.pallas import tpu as pltpu
from jax.experimental.pallas import tpu_sc as plsc

N = 10000
E = 320000
DV = 128
DE = 16
DH = 64
D3 = 3 * DH

EDGE_BLOCK = 2000
NSLICE = 4
ES = E // NSLICE

SC_CORES = 2
SC_SUBCORES = 16
NW = SC_CORES * SC_SUBCORES
CH = 128
ZCH = 40
N_ZCH = N // ZCH


def _rn16(u):
    return u + jnp.uint32(0x8000)


def _proj_body(x_ref, w_ref, b_ref, o_ref):
    t = jnp.dot(x_ref[...], w_ref[...],
                preferred_element_type=jnp.float32) + b_ref[...]
    a = t[:, :DV]
    b = t[:, DV:]
    ba = lax.bitcast_convert_type(a, jnp.uint32)
    bb = lax.bitcast_convert_type(b, jnp.uint32)
    word = (_rn16(ba) >> 16) | (_rn16(bb) & jnp.uint32(0xFFFF0000))
    o_ref[...] = lax.bitcast_convert_type(word, jnp.float32)


def _node_tables(node_fea, Wqkv, bqkv):
    return pl.pallas_call(
        _proj_body,
        out_shape=jax.ShapeDtypeStruct((N, DV), jnp.float32),
    )(node_fea, Wqkv, bqkv)


def _make_gather_body(n_chunks):
    def _gather_body(tp_hbm, idx1_hbm, idx2_hbm, g1_hbm, g2_hbm,
                     i1a, i2a, r1a, r2a, i1b, i2b, r1b, r2b, sema, semb):
        c = lax.axis_index("c")
        s = lax.axis_index("s")
        wid = c * SC_SUBCORES + s
        n_iter = (n_chunks + NW - 1) // NW
        n_pair = (n_iter + 1) // 2

        def _one(u, parity, i1v, i2v, r1v, r2v, sem):
            cid = wid + (2 * u + parity) * NW

            @pl.when(cid < n_chunks)
            def _do():
                base = pl.multiple_of(cid * CH, 8)

                @pl.when(u > 0)
                def _drain():
                    pltpu.make_async_copy(
                        r1v, g1_hbm.at[pl.ds(0, CH)], sem).wait()
                    pltpu.make_async_copy(
                        r2v, g2_hbm.at[pl.ds(0, CH)], sem).wait()

                pltpu.sync_copy(idx1_hbm.at[pl.ds(base, CH)], i1v)
                pltpu.sync_copy(idx2_hbm.at[pl.ds(base, CH)], i2v)
                pltpu.sync_copy(tp_hbm.at[i1v], r1v)
                pltpu.sync_copy(tp_hbm.at[i2v], r2v)
                pltpu.async_copy(r1v, g1_hbm.at[pl.ds(base, CH)], sem)
                pltpu.async_copy(r2v, g2_hbm.at[pl.ds(base, CH)], sem)

        def _pair(u, carry):
            _one(u, 0, i1a, i2a, r1a, r2a, sema)
            _one(u, 1, i1b, i2b, r1b, r2b, semb)
            return carry

        lax.fori_loop(0, n_pair, _pair, 0)
        pltpu.make_async_copy(r1a, g1_hbm.at[pl.ds(0, CH)], sema).wait()
        pltpu.make_async_copy(r2a, g2_hbm.at[pl.ds(0, CH)], sema).wait()
        pltpu.make_async_copy(r1b, g1_hbm.at[pl.ds(0, CH)], semb).wait()
        pltpu.make_async_copy(r2b, g2_hbm.at[pl.ds(0, CH)], semb).wait()

    return _gather_body


def _sc_gather(TP, idx1, idx2):
    ne = idx1.shape[0]
    mesh = plsc.VectorSubcoreMesh(core_axis_name="c", subcore_axis_name="s",
                                  num_cores=SC_CORES, num_subcores=SC_SUBCORES)
    return pl.kernel(
        _make_gather_body(ne // CH),
        out_type=(jax.ShapeDtypeStruct((ne, DV), jnp.float32),
                  jax.ShapeDtypeStruct((ne, DV), jnp.float32)),
        mesh=mesh,
        scratch_types=[
            pltpu.VMEM((CH,), jnp.int32),
            pltpu.VMEM((CH,), jnp.int32),
            pltpu.VMEM((CH, DV), jnp.float32),
            pltpu.VMEM((CH, DV), jnp.float32),
            pltpu.VMEM((CH,), jnp.int32),
            pltpu.VMEM((CH,), jnp.int32),
            pltpu.VMEM((CH, DV), jnp.float32),
            pltpu.VMEM((CH, DV), jnp.float32),
            pltpu.SemaphoreType.DMA,
            pltpu.SemaphoreType.DMA,
        ],
    )(TP, idx1, idx2)


def _ln(x, g, b):
    mu = jnp.mean(x, axis=-1, keepdims=True)
    xc = x - mu
    var = jnp.mean(xc * xc, axis=-1, keepdims=True)
    return xc * jax.lax.rsqrt(var + 1e-5) * g + b


def _edge_body(g1_ref, g2_ref, ef_ref, we_ref, be_ref, g1v_ref, b1v_ref,
               wu_ref, bu_ref, wm_ref, bm_ref, g2v_ref, b2v_ref, o_ref):
    w1 = lax.bitcast_convert_type(g1_ref[...], jnp.uint32)
    w2 = lax.bitcast_convert_type(g2_ref[...], jnp.uint32)
    hi_mask = jnp.uint32(0xFFFF0000)
    QK1 = lax.bitcast_convert_type(w1 << 16, jnp.float32)
    Vx1 = lax.bitcast_convert_type(w1 & hi_mask, jnp.float32)
    QK2 = lax.bitcast_convert_type(w2 << 16, jnp.float32)
    Vx2 = lax.bitcast_convert_type(w2 & hi_mask, jnp.float32)
    Q1 = QK1[:, 0:DH]
    K1 = QK1[:, DH:2 * DH]
    V1 = Vx1[:, 0:DH]
    K2 = QK2[:, DH:2 * DH]
    V2 = Vx2[:, 0:DH]
    EP = jnp.dot(ef_ref[...], we_ref[...],
                 preferred_element_type=jnp.float32) + be_ref[...]
    inv_sqrt = 1.0 / (D3 ** 0.5)
    aij = jnp.concatenate([Q1 * K1, Q1 * K2, Q1 * EP], axis=1) * inv_sqrt
    ln1 = _ln(aij, g1v_ref[...], b1v_ref[...])
    m1 = 1.0 / (1.0 + jnp.exp(-ln1))
    zij = jnp.concatenate([V1, V2, EP], axis=1).astype(jnp.bfloat16)
    m2 = jnp.dot(zij, wu_ref[...].astype(jnp.bfloat16),
                 preferred_element_type=jnp.float32) + bu_ref[...]
    mij = (m1 * m2).astype(jnp.bfloat16)
    t = jnp.dot(mij, wm_ref[...].astype(jnp.bfloat16),
                preferred_element_type=jnp.float32) + bm_ref[...]
    o_ref[...] = _ln(t, g2v_ref[...], b2v_ref[...])


def _edge_mlp(G1, G2, edge_fea, We, be, g1, b1, Wu, bu, Wm, bm, g2, b2):
    ne = G1.shape[0]
    nblk = ne // EDGE_BLOCK
    row_spec = lambda d: pl.BlockSpec((EDGE_BLOCK, d), lambda i: (i, 0))
    full = lambda a, b: pl.BlockSpec((a, b), lambda i: (0, 0))
    return pl.pallas_call(
        _edge_body,
        grid=(nblk,),
        in_specs=[
            row_spec(DV), row_spec(DV), row_spec(DE),
            full(DE, DH), full(1, DH), full(1, D3), full(1, D3),
            full(D3, D3), full(1, D3), full(D3, DV), full(1, DV),
            full(1, DV), full(1, DV),
        ],
        out_specs=row_spec(DV),
        out_shape=jax.ShapeDtypeStruct((ne, DV), jnp.float32),
    )(G1, G2, edge_fea, We, be.reshape(1, DH), g1.reshape(1, D3),
      b1.reshape(1, D3), Wu, bu.reshape(1, D3), Wm, bm.reshape(1, DV),
      g2.reshape(1, DV), b2.reshape(1, DV))


def _make_scatter_body(n_chunks):
    def _scatter_body(msg_hbm, idx_hbm, p0_hbm, p1_hbm,
                      idx_v, msg_v, idx_vb, msg_vb, stage_v, acc,
                      sem_a, sem_b):
        c = lax.axis_index("c")
        s = lax.axis_index("s")
        wid = c * SC_SUBCORES + s

        z16 = jnp.zeros((16,), jnp.float32)

        def _zrow(i, carry):
            for j in range(DV // 16):
                stage_v[i, pl.ds(j * 16, 16)] = z16
            return carry

        lax.fori_loop(0, ZCH, _zrow, 0)

        def _zchunk(t, carry):
            zc = s + t * SC_SUBCORES

            @pl.when(zc < N_ZCH)
            def _do():
                rows = pl.ds(pl.multiple_of(zc * ZCH, 8), ZCH)
                pltpu.sync_copy(stage_v, acc.at[rows])

            return carry

        lax.fori_loop(0, (N_ZCH + SC_SUBCORES - 1) // SC_SUBCORES, _zchunk, 0)
        plsc.subcore_barrier()

        n_iter = (n_chunks + NW - 1) // NW
        n_pair = (n_iter + 1) // 2

        def _one(u, parity, iv, mv, sem):
            cid = wid + (2 * u + parity) * NW

            @pl.when(cid < n_chunks)
            def _do():
                base = pl.multiple_of(cid * CH, 8)

                @pl.when(u > 0)
                def _drain():
                    pltpu.make_async_copy(
                        mv, acc.at[pl.ds(0, CH)], sem).wait()

                pltpu.sync_copy(idx_hbm.at[pl.ds(base, CH)], iv)
                pltpu.sync_copy(msg_hbm.at[pl.ds(base, CH)], mv)
                pltpu.async_copy(mv, acc.at[iv], sem, add=True)

        def _pair(u, carry):
            _one(u, 0, idx_v, msg_v, sem_a)
            _one(u, 1, idx_vb, msg_vb, sem_b)
            return carry

        lax.fori_loop(0, n_pair, _pair, 0)
        pltpu.make_async_copy(msg_v, acc.at[pl.ds(0, CH)], sem_a).wait()
        pltpu.make_async_copy(msg_vb, acc.at[pl.ds(0, CH)], sem_b).wait()
        plsc.subcore_barrier()

        def _dchunk(t, carry):
            zc = s + t * SC_SUBCORES

            @pl.when(zc < N_ZCH)
            def _do():
                rows = pl.ds(pl.multiple_of(zc * ZCH, 8), ZCH)
                pltpu.sync_copy(acc.at[rows], stage_v)

                @pl.when(c == 0)
                def _c0():
                    pltpu.sync_copy(stage_v, p0_hbm.at[rows])

                @pl.when(c == 1)
                def _c1():
                    pltpu.sync_copy(stage_v, p1_hbm.at[rows])

            return carry

        lax.fori_loop(0, (N_ZCH + SC_SUBCORES - 1) // SC_SUBCORES, _dchunk, 0)

    return _scatter_body


def _sc_scatter(msg, idx1):
    ne = idx1.shape[0]
    mesh = plsc.VectorSubcoreMesh(core_axis_name="c", subcore_axis_name="s",
                                  num_cores=SC_CORES, num_subcores=SC_SUBCORES)
    return pl.kernel(
        _make_scatter_body(ne // CH),
        out_type=(jax.ShapeDtypeStruct((N, DV), jnp.float32),
                  jax.ShapeDtypeStruct((N, DV), jnp.float32)),
        mesh=mesh,
        scratch_types=[
            pltpu.VMEM((CH,), jnp.int32),
            pltpu.VMEM((CH, DV), jnp.float32),
            pltpu.VMEM((CH,), jnp.int32),
            pltpu.VMEM((CH, DV), jnp.float32),
            pltpu.VMEM((ZCH, DV), jnp.float32),
            pltpu.VMEM_SHARED((N, DV), jnp.float32),
            pltpu.SemaphoreType.DMA,
            pltpu.SemaphoreType.DMA,
        ],
    )(msg, idx1)


def _add_body(*refs):
    o_ref = refs[-1]
    acc = refs[0][...]
    for r in refs[1:-1]:
        acc = acc + r[...]
    o_ref[...] = acc


def _add_partials(parts):
    spec = pl.BlockSpec((EDGE_BLOCK, DV), lambda i: (i, 0))
    return pl.pallas_call(
        _add_body,
        grid=(N // EDGE_BLOCK,),
        in_specs=[spec] * len(parts),
        out_specs=spec,
        out_shape=jax.ShapeDtypeStruct((N, DV), jnp.float32),
    )(*parts)


def kernel(node_fea, idx1, idx2, edge_fea, Wq, bq, Wk, bk, Wv, bv, We, be,
           g1, b1, Wu, bu, Wm, bm, g2, b2):
    Wqkv = jnp.concatenate(
        [Wq, Wk, Wv, jnp.zeros((DV, 2 * DV - D3), jnp.float32)], axis=1)
    bqkv = jnp.concatenate(
        [bq, bk, bv, jnp.zeros((2 * DV - D3,), jnp.float32)]).reshape(1, 2 * DV)
    TP = _node_tables(node_fea, Wqkv, bqkv)
    parts = []
    for si in range(NSLICE):
        sl = slice(si * ES, (si + 1) * ES)
        i1 = idx1[sl]
        G1, G2 = _sc_gather(TP, i1, idx2[sl])
        msg = _edge_mlp(G1, G2, edge_fea[sl], We, be, g1, b1,
                        Wu, bu, Wm, bm, g2, b2)
        p0, p1 = _sc_scatter(msg, i1)
        parts.extend([p0, p1])
    return _add_partials(parts)
